# baseline (speedup 1.0000x reference)
import functools

import jax
import jax.numpy as jnp
from jax import lax
from jax.experimental import pallas as pl
from jax.experimental.pallas import tpu as pltpu
from jax.experimental.pallas import tpu_sc as plsc

N = 10000
NPAD = 10240
NW = 32
VPW = NPAD // NW
E2 = 640000
CH = 128
CPW = 160
E2P = NW * CPW * CH
D = 128

LEVELS = [(56, 3136, 0), (28, 784, 3136), (14, 196, 3920), (7, 49, 4120)]
GROWS = 4176
CH_SPLITS = [(0, 256), (256, 512), (768, 1024), (1792, 2048)]

@functools.cache
def _get_mesh():
    return plsc.VectorSubcoreMesh(core_axis_name="c", subcore_axis_name="s")


def _g_table_body(f0, f1, f2, f3, w, g_out):
    fs = [f0, f1, f2, f3]
    for (lw, hw, off), (c0, cn), f in zip(LEVELS, CH_SPLITS, fs):
        hwp = f.shape[1]
        gl = lax.dot_general(f[...], w[pl.ds(c0, cn), :],
                             (((0,), (0,)), ((), ())),
                             preferred_element_type=jnp.float32)
        g_out[pl.ds(off, hwp), :] = gl


def _idx_wgt_body(verts, idx_out, wgt_out):
    vx = verts[:, 0:1]
    vy = verts[:, 1:2]
    idx_cols = []
    wgt_cols = []
    for (lw, hw, off), _ in zip(LEVELS, CH_SPLITS):
        x = jnp.clip((vx + 1.0) * 0.5 * (lw - 1), 0.0, lw - 1.0)
        y = jnp.clip((vy + 1.0) * 0.5 * (lw - 1), 0.0, lw - 1.0)
        x0 = jnp.floor(x)
        y0 = jnp.floor(y)
        wx1 = x - x0
        wx0 = 1.0 - wx1
        wy1 = y - y0
        wy0 = 1.0 - wy1
        x0c = x0.astype(jnp.int32)
        x1c = jnp.minimum(x0c + 1, lw - 1)
        y0c = y0.astype(jnp.int32)
        y1c = jnp.minimum(y0c + 1, lw - 1)
        idx_cols += [off + y0c * lw + x0c, off + y0c * lw + x1c,
                     off + y1c * lw + x0c, off + y1c * lw + x1c]
        wgt_cols += [wy0 * wx0, wy0 * wx1, wy1 * wx0, wy1 * wx1]
    idx_out[...] = jnp.concatenate(idx_cols, axis=1)
    wgt_out[...] = jnp.concatenate(wgt_cols, axis=1)


def _g_table(f0, f1, f2, f3, w, verts_p):
    g = pl.pallas_call(
        _g_table_body,
        out_shape=jax.ShapeDtypeStruct((GROWS, D), jnp.float32),
    )(f0, f1, f2, f3, w)
    blk = 512
    idx16, wgt16 = pl.pallas_call(
        _idx_wgt_body,
        grid=(NPAD // blk,),
        in_specs=[pl.BlockSpec((blk, 3), lambda i: (i, 0))],
        out_specs=[pl.BlockSpec((blk, 16), lambda i: (i, 0))] * 2,
        out_shape=[jax.ShapeDtypeStruct((NPAD, 16), jnp.int32),
                   jax.ShapeDtypeStruct((NPAD, 16), jnp.float32)],
    )(verts_p)
    return g, idx16, wgt16


RPW = VPW // 8


def _bilinear_body(g_hbm, idx_hbm, wgt_hbm, b_hbm, out_hbm,
                   idx_v, wgt_v, rows_a, rows_b, out_v, b_v, sem_a, sem_b):
    wid = lax.axis_index("s") * 2 + lax.axis_index("c")
    base = wid * RPW
    pltpu.sync_copy(idx_hbm.at[pl.ds(base, RPW)], idx_v)
    pltpu.sync_copy(wgt_hbm.at[pl.ds(base, RPW)], wgt_v)
    pltpu.sync_copy(b_hbm, b_v)

    bufs = (rows_a, rows_b)
    sems = (sem_a, sem_b)

    def compute(r, buf):
        for v8 in range(8):
            wchunk = wgt_v[r, pl.ds(v8 * 16, 16)]
            for c in range(8):
                acc = b_v[pl.ds(c * 16, 16)]
                for j in range(16):
                    acc = acc + wchunk[j] * buf[v8 * 16 + j, pl.ds(c * 16, 16)]
                out_v[8 * r + v8, pl.ds(c * 16, 16)] = jnp.maximum(acc, 0.0)

    def wait(r, k):
        pltpu.make_async_copy(g_hbm.at[idx_v.at[r]], bufs[k], sems[k]).wait()

    pltpu.async_copy(g_hbm.at[idx_v.at[0]], rows_a, sem_a)

    def body(t, _):
        r0 = 2 * t
        pltpu.async_copy(g_hbm.at[idx_v.at[r0 + 1]], rows_b, sem_b)
        wait(r0, 0)
        compute(r0, rows_a)

        @pl.when(t < RPW // 2 - 1)
        def _():
            pltpu.async_copy(g_hbm.at[idx_v.at[r0 + 2]], rows_a, sem_a)

        wait(r0 + 1, 1)
        compute(r0 + 1, rows_b)
        return ()

    lax.fori_loop(0, RPW // 2, body, ())
    pltpu.sync_copy(out_v, out_hbm.at[pl.ds(wid * VPW, VPW)])


@functools.cache
def _get_bilinear():
    return pl.kernel(
        _bilinear_body,
        out_type=jax.ShapeDtypeStruct((NPAD, D), jnp.float32),
        mesh=_get_mesh(),
        scratch_types=[
            pltpu.VMEM((RPW, CH), jnp.int32),
            pltpu.VMEM((RPW, CH), jnp.float32),
            pltpu.VMEM((CH, D), jnp.float32),
            pltpu.VMEM((CH, D), jnp.float32),
            pltpu.VMEM((VPW, D), jnp.float32),
            pltpu.VMEM((D,), jnp.float32),
            pltpu.SemaphoreType.DMA,
            pltpu.SemaphoreType.DMA,
        ],
    )


IB = 40


def _scatter_body(vw_hbm, gidx_hbm, sidx_hbm, zeros_hbm, out_hbm,
                  gidx_v, sidx_v, rows_a, rows_b, acc_sh,
                  sem_ga, sem_gb, sem_sa, sem_sb):
    cid = lax.axis_index("c")
    sid = lax.axis_index("s")
    wid = sid * 2 + cid
    rpt = NPAD // 16
    r0 = sid * rpt
    pltpu.sync_copy(zeros_hbm.at[pl.ds(r0, rpt)], acc_sh.at[pl.ds(r0, rpt)])
    plsc.subcore_barrier()

    bufs = (rows_a, rows_b)
    gsems = (sem_ga, sem_gb)
    ssems = (sem_sa, sem_sb)

    def outer(jb, _):
        c0 = wid * CPW + jb * IB
        pltpu.sync_copy(gidx_hbm.at[pl.ds(c0, IB)], gidx_v)
        pltpu.sync_copy(sidx_hbm.at[pl.ds(c0, IB)], sidx_v)
        gd = [None, None]
        sd = [None, None]
        gd[0] = pltpu.async_copy(vw_hbm.at[gidx_v.at[0]], bufs[0], gsems[0])
        for q in range(IB):
            x = q % 2
            y = (q + 1) % 2
            if q + 1 < IB:
                if sd[y] is not None:
                    sd[y].wait()
                gd[y] = pltpu.async_copy(
                    vw_hbm.at[gidx_v.at[q + 1]], bufs[y], gsems[y])
            gd[x].wait()
            sd[x] = pltpu.async_copy(
                bufs[x], acc_sh.at[sidx_v.at[q]], ssems[x], add=True)
        sd[0].wait()
        sd[1].wait()
        return ()

    lax.fori_loop(0, CPW // IB, outer, ())
    plsc.subcore_barrier()
    pltpu.sync_copy(acc_sh.at[pl.ds(r0, rpt)],
                    out_hbm.at[pl.ds(cid * NPAD + r0, rpt)])


@functools.cache
def _make_scatter(dd, tc_tiling=True):
    return pl.kernel(
        _scatter_body,
        out_type=jax.ShapeDtypeStruct((2 * NPAD, dd), jnp.float32),
        mesh=_get_mesh(),
        compiler_params=pltpu.CompilerParams(use_tc_tiling_on_sc=tc_tiling),
        scratch_types=[
            pltpu.VMEM((IB, CH), jnp.int32),
            pltpu.VMEM((IB, CH), jnp.int32),
            pltpu.VMEM((CH, dd), jnp.float32),
            pltpu.VMEM((CH, dd), jnp.float32),
            pltpu.VMEM_SHARED((NPAD, dd), jnp.float32),
            pltpu.SemaphoreType.DMA,
            pltpu.SemaphoreType.DMA,
            pltpu.SemaphoreType.DMA,
            pltpu.SemaphoreType.DMA,
        ],
    )


BLK = 2048


def _mm_body(n_in, d_out, relu, emit_sum, *refs):
    ins = refs[:n_in]
    w_ref, b_ref = refs[n_in], refs[n_in + 1]
    outs = refs[n_in + 2:]
    x = ins[0][...]
    for r in ins[1:]:
        x = x + r[...]
    if emit_sum:
        outs[2][...] = x
    a = jnp.maximum(x, 0.0) if relu else x
    u = jnp.dot(a, w_ref[...], preferred_element_type=jnp.float32) + b_ref[...]
    h = d_out // 2
    outs[0][...] = u[:, :h]
    outs[1][...] = u[:, h:]


def _combine_matmul(addends, w, b, relu, emit_sum):
    n_in = len(addends)
    d_out = w.shape[1]
    h = d_out // 2
    grid = NPAD // BLK
    in_specs = ([pl.BlockSpec((BLK, D), lambda i: (i, 0)) for _ in addends]
                + [pl.BlockSpec((D, d_out), lambda i: (0, 0)),
                   pl.BlockSpec((1, d_out), lambda i: (0, 0))])
    out_shape = [jax.ShapeDtypeStruct((NPAD, h), jnp.float32),
                 jax.ShapeDtypeStruct((NPAD, h), jnp.float32)]
    out_specs = [pl.BlockSpec((BLK, h), lambda i: (i, 0)),
                 pl.BlockSpec((BLK, h), lambda i: (i, 0))]
    if emit_sum:
        out_shape.append(jax.ShapeDtypeStruct((NPAD, D), jnp.float32))
        out_specs.append(pl.BlockSpec((BLK, D), lambda i: (i, 0)))
    return pl.pallas_call(
        functools.partial(_mm_body, n_in, d_out, relu, emit_sum),
        grid=(grid,),
        in_specs=in_specs,
        out_specs=out_specs,
        out_shape=out_shape,
    )(*addends, w, b.reshape(1, d_out))


def _first_mm_body(img, verts, wci, wcv, bc, wsi, wsv, bs,
                   out1, vw1, sk):
    rv = jnp.maximum(verts[...], 0.0)
    u = (jnp.dot(img[...], wci[...], preferred_element_type=jnp.float32)
         + jnp.dot(rv, wcv[...], preferred_element_type=jnp.float32)
         + bc[...])
    out1[...] = u[:, :D]
    vw1[...] = u[:, D:]
    sk[...] = (jnp.dot(img[...], wsi[...], preferred_element_type=jnp.float32)
               + jnp.dot(verts[...], wsv[...], preferred_element_type=jnp.float32)
               + bs[...])


def _first_mm(img, verts_p, wci, wcv, bc, wsi, wsv, bs):
    grid = NPAD // BLK
    return pl.pallas_call(
        _first_mm_body,
        grid=(grid,),
        in_specs=[
            pl.BlockSpec((BLK, D), lambda i: (i, 0)),
            pl.BlockSpec((BLK, 3), lambda i: (i, 0)),
            pl.BlockSpec((D, 2 * D), lambda i: (0, 0)),
            pl.BlockSpec((3, 2 * D), lambda i: (0, 0)),
            pl.BlockSpec((1, 2 * D), lambda i: (0, 0)),
            pl.BlockSpec((D, D), lambda i: (0, 0)),
            pl.BlockSpec((3, D), lambda i: (0, 0)),
            pl.BlockSpec((1, D), lambda i: (0, 0)),
        ],
        out_specs=[pl.BlockSpec((BLK, D), lambda i: (i, 0))] * 3,
        out_shape=[jax.ShapeDtypeStruct((NPAD, D), jnp.float32)] * 3,
    )(img, verts_p, wci, wcv, bc.reshape(1, 2 * D), wsi, wsv, bs.reshape(1, D))


def _final_body(verts, o16, ma, mb, out):
    t = o16[:, :3] + ma[:, :3] + mb[:, :3]
    out[...] = verts[...] + jnp.tanh(t)


def _final(verts_p, o16, ma, mb):
    grid = NPAD // BLK
    return pl.pallas_call(
        _final_body,
        grid=(grid,),
        in_specs=[
            pl.BlockSpec((BLK, 3), lambda i: (i, 0)),
            pl.BlockSpec((BLK, 16), lambda i: (i, 0)),
            pl.BlockSpec((BLK, 16), lambda i: (i, 0)),
            pl.BlockSpec((BLK, 16), lambda i: (i, 0)),
        ],
        out_specs=pl.BlockSpec((BLK, 3), lambda i: (i, 0)),
        out_shape=jax.ShapeDtypeStruct((NPAD, 3), jnp.float32),
    )(verts_p, o16, ma, mb)


def kernel(feats0, feats1, feats2, feats3, verts, params, edges):
    p = params
    f32 = jnp.float32

    f0 = feats0[0].reshape(256, 3136)
    f1 = feats1[0].reshape(512, 784)
    f2 = jnp.pad(feats2[0].reshape(1024, 196), ((0, 0), (0, 4)))
    f3 = jnp.pad(feats3[0].reshape(2048, 49), ((0, 0), (0, 7)))
    verts_p = jnp.pad(verts, ((0, NPAD - N), (0, 0)))

    s = edges[:, 0].astype(jnp.int32)
    d = edges[:, 1].astype(jnp.int32)
    gidx = jnp.concatenate([d, s])
    sidx = jnp.concatenate([s, d])
    padn = E2P - E2
    gidx = jnp.concatenate(
        [gidx, jnp.arange(padn, dtype=jnp.int32) % N])
    sidx = jnp.concatenate(
        [sidx, N + (jnp.arange(padn, dtype=jnp.int32) % (NPAD - N))])
    gidx = gidx.reshape(NW * CPW, CH)
    sidx = sidx.reshape(NW * CPW, CH)
    zeros128 = jnp.zeros((NPAD, D), f32)

    def wcat(pre):
        return (jnp.concatenate([p[pre + "_w0"], p[pre + "_w1"]], axis=1),
                jnp.concatenate([p[pre + "_b0"], p[pre + "_b1"]]))

    g, idx16, wgt16 = _g_table(f0, f1, f2, f3, p["bottleneck_w"], verts_p)

    img = _get_bilinear()(g, idx16.reshape(NPAD // 8, 128),
                          wgt16.reshape(NPAD // 8, 128), p["bottleneck_b"])

    def scat128(vw):
        n = _make_scatter(D)(vw, gidx, sidx, zeros128)
        return n[:NPAD], n[NPAD:]

    w1c, b1c = wcat("rb0_gc1")
    out1, vw1, sk = _first_mm(img, verts_p, w1c[:D], w1c[D:], b1c,
                              p["rb0_skip_w"][:D], p["rb0_skip_w"][D:],
                              p["rb0_skip_b"])
    na, nb = scat128(vw1)
    w2c, b2c = wcat("rb0_gc2")
    out2, vw2 = _combine_matmul([out1, na, nb], w2c, b2c, True, False)
    na, nb = scat128(vw2)

    w3c, b3c = wcat("rb1_gc1")
    out3, vw3, x1 = _combine_matmul([out2, na, nb, sk], w3c, b3c, True, True)
    na, nb = scat128(vw3)
    w4c, b4c = wcat("rb1_gc2")
    out4, vw4 = _combine_matmul([out3, na, nb], w4c, b4c, True, False)
    na, nb = scat128(vw4)

    w5c, b5c = wcat("rb2_gc1")
    out5, vw5, x2 = _combine_matmul([out4, na, nb, x1], w5c, b5c, True, True)
    na, nb = scat128(vw5)
    w6c, b6c = wcat("rb2_gc2")
    out6, vw6 = _combine_matmul([out5, na, nb], w6c, b6c, True, False)
    na, nb = scat128(vw6)

    wo = jnp.concatenate([
        jnp.pad(p["out_w0"], ((0, 0), (0, 13))),
        jnp.pad(p["out_w1"], ((0, 0), (0, 13)))], axis=1)
    bo = jnp.concatenate([jnp.pad(p["out_b0"], (0, 13)),
                          jnp.pad(p["out_b1"], (0, 13))])
    o16, vwo, x3 = _combine_matmul([out6, na, nb, x2], wo, bo, False, True)
    zeros16 = jnp.zeros((NPAD, 16), jnp.float32)
    m = _make_scatter(16, False)(vwo, gidx, sidx, zeros16)
    new_verts = _final(verts_p, o16, m[:NPAD], m[NPAD:])

    return new_verts[:N], x3[:N]


# speedup vs baseline: 1.0173x; 1.0173x over previous
import functools

import jax
import jax.numpy as jnp
from jax import lax
from jax.experimental import pallas as pl
from jax.experimental.pallas import tpu as pltpu
from jax.experimental.pallas import tpu_sc as plsc

N = 10000
NPAD = 10240
NW = 32
VPW = NPAD // NW
E2 = 640000
CH = 128
CPW = 160
E2P = NW * CPW * CH
D = 128

LEVELS = [(56, 3136, 0), (28, 784, 3136), (14, 196, 3920), (7, 49, 4120)]
GROWS = 4176
CH_SPLITS = [(0, 256), (256, 512), (768, 1024), (1792, 2048)]

@functools.cache
def _get_mesh():
    return plsc.VectorSubcoreMesh(core_axis_name="c", subcore_axis_name="s")


def _g_table_body(f0, f1, f2, f3, w, g_out):
    fs = [f0, f1, f2, f3]
    for (lw, hw, off), (c0, cn), f in zip(LEVELS, CH_SPLITS, fs):
        hwp = f.shape[1]
        gl = lax.dot_general(f[...], w[pl.ds(c0, cn), :],
                             (((0,), (0,)), ((), ())),
                             preferred_element_type=jnp.float32)
        g_out[pl.ds(off, hwp), :] = gl


def _idx_wgt_body(verts, idx_out, wgt_out):
    vx = verts[:, 0:1]
    vy = verts[:, 1:2]
    idx_cols = []
    wgt_cols = []
    for (lw, hw, off), _ in zip(LEVELS, CH_SPLITS):
        x = jnp.clip((vx + 1.0) * 0.5 * (lw - 1), 0.0, lw - 1.0)
        y = jnp.clip((vy + 1.0) * 0.5 * (lw - 1), 0.0, lw - 1.0)
        x0 = jnp.floor(x)
        y0 = jnp.floor(y)
        wx1 = x - x0
        wx0 = 1.0 - wx1
        wy1 = y - y0
        wy0 = 1.0 - wy1
        x0c = x0.astype(jnp.int32)
        x1c = jnp.minimum(x0c + 1, lw - 1)
        y0c = y0.astype(jnp.int32)
        y1c = jnp.minimum(y0c + 1, lw - 1)
        idx_cols += [off + y0c * lw + x0c, off + y0c * lw + x1c,
                     off + y1c * lw + x0c, off + y1c * lw + x1c]
        wgt_cols += [wy0 * wx0, wy0 * wx1, wy1 * wx0, wy1 * wx1]
    idx_out[...] = jnp.concatenate(idx_cols, axis=1)
    wgt_out[...] = jnp.concatenate(wgt_cols, axis=1)


def _g_table(f0, f1, f2, f3, w, verts_p):
    g = pl.pallas_call(
        _g_table_body,
        out_shape=jax.ShapeDtypeStruct((GROWS, D), jnp.float32),
    )(f0, f1, f2, f3, w)
    blk = 512
    idx16, wgt16 = pl.pallas_call(
        _idx_wgt_body,
        grid=(NPAD // blk,),
        in_specs=[pl.BlockSpec((blk, 3), lambda i: (i, 0))],
        out_specs=[pl.BlockSpec((blk, 16), lambda i: (i, 0))] * 2,
        out_shape=[jax.ShapeDtypeStruct((NPAD, 16), jnp.int32),
                   jax.ShapeDtypeStruct((NPAD, 16), jnp.float32)],
    )(verts_p)
    return g, idx16, wgt16


def _bilinear_body(g_hbm, idx_hbm, wgt_hbm, b_hbm, out_hbm,
                   idx_v, wgt_v, rows_a, rows_b, out_v, b_v, sem_a, sem_b):
    wid = lax.axis_index("s") * 2 + lax.axis_index("c")
    base = wid * VPW
    pltpu.sync_copy(idx_hbm.at[pl.ds(base, VPW)], idx_v)
    pltpu.sync_copy(wgt_hbm.at[pl.ds(base, VPW)], wgt_v)
    pltpu.sync_copy(b_hbm, b_v)

    bufs = (rows_a, rows_b)
    sems = (sem_a, sem_b)

    def compute(v, buf):
        wrow = wgt_v[v]
        for c in range(8):
            acc = b_v[pl.ds(c * 16, 16)]
            for j in range(16):
                acc = acc + wrow[j] * buf[j, pl.ds(c * 16, 16)]
            out_v[v, pl.ds(c * 16, 16)] = jnp.maximum(acc, 0.0)

    def wait(v, k):
        pltpu.make_async_copy(g_hbm.at[idx_v[v]], bufs[k], sems[k]).wait()

    pltpu.async_copy(g_hbm.at[idx_v[0]], rows_a, sem_a)

    def body(t, _):
        v0 = 2 * t
        pltpu.async_copy(g_hbm.at[idx_v[v0 + 1]], rows_b, sem_b)
        wait(v0, 0)
        compute(v0, rows_a)

        @pl.when(t < VPW // 2 - 1)
        def _():
            pltpu.async_copy(g_hbm.at[idx_v[v0 + 2]], rows_a, sem_a)

        wait(v0 + 1, 1)
        compute(v0 + 1, rows_b)
        return ()

    lax.fori_loop(0, VPW // 2, body, ())
    pltpu.sync_copy(out_v, out_hbm.at[pl.ds(base, VPW)])


@functools.cache
def _get_bilinear():
    return pl.kernel(
        _bilinear_body,
        out_type=jax.ShapeDtypeStruct((NPAD, D), jnp.float32),
        mesh=_get_mesh(),
        scratch_types=[
            pltpu.VMEM((VPW, 16), jnp.int32),
            pltpu.VMEM((VPW, 16), jnp.float32),
            pltpu.VMEM((16, D), jnp.float32),
            pltpu.VMEM((16, D), jnp.float32),
            pltpu.VMEM((VPW, D), jnp.float32),
            pltpu.VMEM((D,), jnp.float32),
            pltpu.SemaphoreType.DMA,
            pltpu.SemaphoreType.DMA,
        ],
    )


IB = 40


def _scatter_body(vw_hbm, gidx_hbm, sidx_hbm, zeros_hbm, out_hbm,
                  gidx_v, sidx_v, rows_a, rows_b, acc_sh,
                  sem_ga, sem_gb, sem_sa, sem_sb):
    cid = lax.axis_index("c")
    sid = lax.axis_index("s")
    wid = sid * 2 + cid
    rpt = NPAD // 16
    r0 = sid * rpt
    pltpu.sync_copy(zeros_hbm.at[pl.ds(r0, rpt)], acc_sh.at[pl.ds(r0, rpt)])
    plsc.subcore_barrier()

    bufs = (rows_a, rows_b)
    gsems = (sem_ga, sem_gb)
    ssems = (sem_sa, sem_sb)

    def outer(jb, _):
        c0 = wid * CPW + jb * IB
        pltpu.sync_copy(gidx_hbm.at[pl.ds(c0, IB)], gidx_v)
        pltpu.sync_copy(sidx_hbm.at[pl.ds(c0, IB)], sidx_v)
        gd = [None, None]
        sd = [None, None]
        gd[0] = pltpu.async_copy(vw_hbm.at[gidx_v.at[0]], bufs[0], gsems[0])
        for q in range(IB):
            x = q % 2
            y = (q + 1) % 2
            if q + 1 < IB:
                if sd[y] is not None:
                    sd[y].wait()
                gd[y] = pltpu.async_copy(
                    vw_hbm.at[gidx_v.at[q + 1]], bufs[y], gsems[y])
            gd[x].wait()
            sd[x] = pltpu.async_copy(
                bufs[x], acc_sh.at[sidx_v.at[q]], ssems[x], add=True)
        sd[0].wait()
        sd[1].wait()
        return ()

    lax.fori_loop(0, CPW // IB, outer, ())
    plsc.subcore_barrier()
    pltpu.sync_copy(acc_sh.at[pl.ds(r0, rpt)],
                    out_hbm.at[pl.ds(cid * NPAD + r0, rpt)])


@functools.cache
def _make_scatter(dd, tc_tiling=True):
    return pl.kernel(
        _scatter_body,
        out_type=jax.ShapeDtypeStruct((2 * NPAD, dd), jnp.float32),
        mesh=_get_mesh(),
        compiler_params=pltpu.CompilerParams(use_tc_tiling_on_sc=tc_tiling),
        scratch_types=[
            pltpu.VMEM((IB, CH), jnp.int32),
            pltpu.VMEM((IB, CH), jnp.int32),
            pltpu.VMEM((CH, dd), jnp.float32),
            pltpu.VMEM((CH, dd), jnp.float32),
            pltpu.VMEM_SHARED((NPAD, dd), jnp.float32),
            pltpu.SemaphoreType.DMA,
            pltpu.SemaphoreType.DMA,
            pltpu.SemaphoreType.DMA,
            pltpu.SemaphoreType.DMA,
        ],
    )


BLK = 2048


def _mm_body(n_in, d_out, relu, emit_sum, *refs):
    ins = refs[:n_in]
    w_ref, b_ref = refs[n_in], refs[n_in + 1]
    outs = refs[n_in + 2:]
    x = ins[0][...]
    for r in ins[1:]:
        x = x + r[...]
    if emit_sum:
        outs[2][...] = x
    a = jnp.maximum(x, 0.0) if relu else x
    u = jnp.dot(a, w_ref[...], preferred_element_type=jnp.float32) + b_ref[...]
    h = d_out // 2
    outs[0][...] = u[:, :h]
    outs[1][...] = u[:, h:]


def _combine_matmul(addends, w, b, relu, emit_sum):
    n_in = len(addends)
    d_out = w.shape[1]
    h = d_out // 2
    grid = NPAD // BLK
    in_specs = ([pl.BlockSpec((BLK, D), lambda i: (i, 0)) for _ in addends]
                + [pl.BlockSpec((D, d_out), lambda i: (0, 0)),
                   pl.BlockSpec((1, d_out), lambda i: (0, 0))])
    out_shape = [jax.ShapeDtypeStruct((NPAD, h), jnp.float32),
                 jax.ShapeDtypeStruct((NPAD, h), jnp.float32)]
    out_specs = [pl.BlockSpec((BLK, h), lambda i: (i, 0)),
                 pl.BlockSpec((BLK, h), lambda i: (i, 0))]
    if emit_sum:
        out_shape.append(jax.ShapeDtypeStruct((NPAD, D), jnp.float32))
        out_specs.append(pl.BlockSpec((BLK, D), lambda i: (i, 0)))
    return pl.pallas_call(
        functools.partial(_mm_body, n_in, d_out, relu, emit_sum),
        grid=(grid,),
        in_specs=in_specs,
        out_specs=out_specs,
        out_shape=out_shape,
    )(*addends, w, b.reshape(1, d_out))


def _first_mm_body(img, verts, wci, wcv, bc, wsi, wsv, bs,
                   out1, vw1, sk):
    rv = jnp.maximum(verts[...], 0.0)
    u = (jnp.dot(img[...], wci[...], preferred_element_type=jnp.float32)
         + jnp.dot(rv, wcv[...], preferred_element_type=jnp.float32)
         + bc[...])
    out1[...] = u[:, :D]
    vw1[...] = u[:, D:]
    sk[...] = (jnp.dot(img[...], wsi[...], preferred_element_type=jnp.float32)
               + jnp.dot(verts[...], wsv[...], preferred_element_type=jnp.float32)
               + bs[...])


def _first_mm(img, verts_p, wci, wcv, bc, wsi, wsv, bs):
    grid = NPAD // BLK
    return pl.pallas_call(
        _first_mm_body,
        grid=(grid,),
        in_specs=[
            pl.BlockSpec((BLK, D), lambda i: (i, 0)),
            pl.BlockSpec((BLK, 3), lambda i: (i, 0)),
            pl.BlockSpec((D, 2 * D), lambda i: (0, 0)),
            pl.BlockSpec((3, 2 * D), lambda i: (0, 0)),
            pl.BlockSpec((1, 2 * D), lambda i: (0, 0)),
            pl.BlockSpec((D, D), lambda i: (0, 0)),
            pl.BlockSpec((3, D), lambda i: (0, 0)),
            pl.BlockSpec((1, D), lambda i: (0, 0)),
        ],
        out_specs=[pl.BlockSpec((BLK, D), lambda i: (i, 0))] * 3,
        out_shape=[jax.ShapeDtypeStruct((NPAD, D), jnp.float32)] * 3,
    )(img, verts_p, wci, wcv, bc.reshape(1, 2 * D), wsi, wsv, bs.reshape(1, D))


def _final_body(verts, o16, ma, mb, out):
    t = o16[:, :3] + ma[:, :3] + mb[:, :3]
    out[...] = verts[...] + jnp.tanh(t)


def _final(verts_p, o16, ma, mb):
    grid = NPAD // BLK
    return pl.pallas_call(
        _final_body,
        grid=(grid,),
        in_specs=[
            pl.BlockSpec((BLK, 3), lambda i: (i, 0)),
            pl.BlockSpec((BLK, 16), lambda i: (i, 0)),
            pl.BlockSpec((BLK, 16), lambda i: (i, 0)),
            pl.BlockSpec((BLK, 16), lambda i: (i, 0)),
        ],
        out_specs=pl.BlockSpec((BLK, 3), lambda i: (i, 0)),
        out_shape=jax.ShapeDtypeStruct((NPAD, 3), jnp.float32),
    )(verts_p, o16, ma, mb)


def kernel(feats0, feats1, feats2, feats3, verts, params, edges):
    p = params
    f32 = jnp.float32

    f0 = feats0[0].reshape(256, 3136)
    f1 = feats1[0].reshape(512, 784)
    f2 = jnp.pad(feats2[0].reshape(1024, 196), ((0, 0), (0, 4)))
    f3 = jnp.pad(feats3[0].reshape(2048, 49), ((0, 0), (0, 7)))
    verts_p = jnp.pad(verts, ((0, NPAD - N), (0, 0)))

    s = edges[:, 0].astype(jnp.int32)
    d = edges[:, 1].astype(jnp.int32)
    gidx = jnp.concatenate([d, s])
    sidx = jnp.concatenate([s, d])
    padn = E2P - E2
    gidx = jnp.concatenate(
        [gidx, jnp.arange(padn, dtype=jnp.int32) % N])
    sidx = jnp.concatenate(
        [sidx, N + (jnp.arange(padn, dtype=jnp.int32) % (NPAD - N))])
    gidx = gidx.reshape(NW * CPW, CH)
    sidx = sidx.reshape(NW * CPW, CH)
    zeros128 = jnp.zeros((NPAD, D), f32)

    def wcat(pre):
        return (jnp.concatenate([p[pre + "_w0"], p[pre + "_w1"]], axis=1),
                jnp.concatenate([p[pre + "_b0"], p[pre + "_b1"]]))

    g, idx16, wgt16 = _g_table(f0, f1, f2, f3, p["bottleneck_w"], verts_p)

    img = _get_bilinear()(g, idx16, wgt16, p["bottleneck_b"])

    def scat128(vw):
        n = _make_scatter(D)(vw, gidx, sidx, zeros128)
        return n[:NPAD], n[NPAD:]

    w1c, b1c = wcat("rb0_gc1")
    out1, vw1, sk = _first_mm(img, verts_p, w1c[:D], w1c[D:], b1c,
                              p["rb0_skip_w"][:D], p["rb0_skip_w"][D:],
                              p["rb0_skip_b"])
    na, nb = scat128(vw1)
    w2c, b2c = wcat("rb0_gc2")
    out2, vw2 = _combine_matmul([out1, na, nb], w2c, b2c, True, False)
    na, nb = scat128(vw2)

    w3c, b3c = wcat("rb1_gc1")
    out3, vw3, x1 = _combine_matmul([out2, na, nb, sk], w3c, b3c, True, True)
    na, nb = scat128(vw3)
    w4c, b4c = wcat("rb1_gc2")
    out4, vw4 = _combine_matmul([out3, na, nb], w4c, b4c, True, False)
    na, nb = scat128(vw4)

    w5c, b5c = wcat("rb2_gc1")
    out5, vw5, x2 = _combine_matmul([out4, na, nb, x1], w5c, b5c, True, True)
    na, nb = scat128(vw5)
    w6c, b6c = wcat("rb2_gc2")
    out6, vw6 = _combine_matmul([out5, na, nb], w6c, b6c, True, False)
    na, nb = scat128(vw6)

    wo = jnp.concatenate([
        jnp.pad(p["out_w0"], ((0, 0), (0, 13))),
        jnp.pad(p["out_w1"], ((0, 0), (0, 13)))], axis=1)
    bo = jnp.concatenate([jnp.pad(p["out_b0"], (0, 13)),
                          jnp.pad(p["out_b1"], (0, 13))])
    o16, vwo, x3 = _combine_matmul([out6, na, nb, x2], wo, bo, False, True)
    zeros16 = jnp.zeros((NPAD, 16), jnp.float32)
    m = _make_scatter(16, False)(vwo, gidx, sidx, zeros16)
    new_verts = _final(verts_p, o16, m[:NPAD], m[NPAD:])

    return new_verts[:N], x3[:N]


# speedup vs baseline: 1.0351x; 1.0175x over previous
import functools

import jax
import jax.numpy as jnp
from jax import lax
from jax.experimental import pallas as pl
from jax.experimental.pallas import tpu as pltpu
from jax.experimental.pallas import tpu_sc as plsc

N = 10000
NPAD = 10240
NW = 32
VPW = NPAD // NW
E2 = 640000
CH = 128
CPW = 160
E2P = NW * CPW * CH
D = 128

LEVELS = [(56, 3136, 0), (28, 784, 3136), (14, 196, 3920), (7, 49, 4120)]
GROWS = 4176
CH_SPLITS = [(0, 256), (256, 512), (768, 1024), (1792, 2048)]

@functools.cache
def _get_mesh():
    return plsc.VectorSubcoreMesh(core_axis_name="c", subcore_axis_name="s")


def _g_table_body(f0, f1, f2, f3, w, g_out):
    fs = [f0, f1, f2, f3]
    for (lw, hw, off), (c0, cn), f in zip(LEVELS, CH_SPLITS, fs):
        hwp = f.shape[1]
        gl = lax.dot_general(f[...], w[pl.ds(c0, cn), :],
                             (((0,), (0,)), ((), ())),
                             preferred_element_type=jnp.float32)
        g_out[pl.ds(off, hwp), :] = gl


def _idx_wgt_body(verts, idx_out, wgt_out):
    vx = verts[:, 0:1]
    vy = verts[:, 1:2]
    idx_cols = []
    wgt_cols = []
    for (lw, hw, off), _ in zip(LEVELS, CH_SPLITS):
        x = jnp.clip((vx + 1.0) * 0.5 * (lw - 1), 0.0, lw - 1.0)
        y = jnp.clip((vy + 1.0) * 0.5 * (lw - 1), 0.0, lw - 1.0)
        x0 = jnp.floor(x)
        y0 = jnp.floor(y)
        wx1 = x - x0
        wx0 = 1.0 - wx1
        wy1 = y - y0
        wy0 = 1.0 - wy1
        x0c = x0.astype(jnp.int32)
        x1c = jnp.minimum(x0c + 1, lw - 1)
        y0c = y0.astype(jnp.int32)
        y1c = jnp.minimum(y0c + 1, lw - 1)
        idx_cols += [off + y0c * lw + x0c, off + y0c * lw + x1c,
                     off + y1c * lw + x0c, off + y1c * lw + x1c]
        wgt_cols += [wy0 * wx0, wy0 * wx1, wy1 * wx0, wy1 * wx1]
    idx_out[...] = jnp.concatenate(idx_cols, axis=1)
    wgt_out[...] = jnp.concatenate(wgt_cols, axis=1)


def _g_table(f0, f1, f2, f3, w, verts_p):
    g = pl.pallas_call(
        _g_table_body,
        out_shape=jax.ShapeDtypeStruct((GROWS, D), jnp.float32),
    )(f0, f1, f2, f3, w)
    blk = 512
    idx16, wgt16 = pl.pallas_call(
        _idx_wgt_body,
        grid=(NPAD // blk,),
        in_specs=[pl.BlockSpec((blk, 3), lambda i: (i, 0))],
        out_specs=[pl.BlockSpec((blk, 16), lambda i: (i, 0))] * 2,
        out_shape=[jax.ShapeDtypeStruct((NPAD, 16), jnp.int32),
                   jax.ShapeDtypeStruct((NPAD, 16), jnp.float32)],
    )(verts_p)
    return g, idx16, wgt16


def _bilinear_body(g_hbm, idx_hbm, wgt_hbm, b_hbm, out_hbm,
                   idx_v, wgt_v, rows_a, rows_b, out_v, b_v, sem_a, sem_b):
    wid = lax.axis_index("s") * 2 + lax.axis_index("c")
    base = wid * VPW
    pltpu.sync_copy(idx_hbm.at[pl.ds(base, VPW)], idx_v)
    pltpu.sync_copy(wgt_hbm.at[pl.ds(base, VPW)], wgt_v)
    pltpu.sync_copy(b_hbm, b_v)

    bufs = (rows_a, rows_b)
    sems = (sem_a, sem_b)

    def compute(v, buf):
        wrow = wgt_v[v]
        for c in range(8):
            acc = b_v[pl.ds(c * 16, 16)]
            for j in range(16):
                acc = acc + wrow[j] * buf[j, pl.ds(c * 16, 16)]
            out_v[v, pl.ds(c * 16, 16)] = jnp.maximum(acc, 0.0)

    def wait(v, k):
        pltpu.make_async_copy(g_hbm.at[idx_v[v]], bufs[k], sems[k]).wait()

    pltpu.async_copy(g_hbm.at[idx_v[0]], rows_a, sem_a)

    def body(t, _):
        v0 = 2 * t
        pltpu.async_copy(g_hbm.at[idx_v[v0 + 1]], rows_b, sem_b)
        wait(v0, 0)
        compute(v0, rows_a)

        @pl.when(t < VPW // 2 - 1)
        def _():
            pltpu.async_copy(g_hbm.at[idx_v[v0 + 2]], rows_a, sem_a)

        wait(v0 + 1, 1)
        compute(v0 + 1, rows_b)
        return ()

    lax.fori_loop(0, VPW // 2, body, ())
    pltpu.sync_copy(out_v, out_hbm.at[pl.ds(base, VPW)])


@functools.cache
def _get_bilinear():
    return pl.kernel(
        _bilinear_body,
        out_type=jax.ShapeDtypeStruct((NPAD, D), jnp.float32),
        mesh=_get_mesh(),
        scratch_types=[
            pltpu.VMEM((VPW, 16), jnp.int32),
            pltpu.VMEM((VPW, 16), jnp.float32),
            pltpu.VMEM((16, D), jnp.float32),
            pltpu.VMEM((16, D), jnp.float32),
            pltpu.VMEM((VPW, D), jnp.float32),
            pltpu.VMEM((D,), jnp.float32),
            pltpu.SemaphoreType.DMA,
            pltpu.SemaphoreType.DMA,
        ],
    )


IB = 40


def _scatter_body(nbuf, vw_hbm, gidx_hbm, sidx_hbm, zeros_hbm, out_hbm,
                  gidx_v, sidx_v, *rest):
    bufs = rest[:nbuf]
    acc_sh = rest[nbuf]
    gsems = rest[nbuf + 1:2 * nbuf + 1]
    ssems = rest[2 * nbuf + 1:]
    cid = lax.axis_index("c")
    sid = lax.axis_index("s")
    wid = sid * 2 + cid
    rpt = NPAD // 16
    r0 = sid * rpt
    pltpu.sync_copy(zeros_hbm.at[pl.ds(r0, rpt)], acc_sh.at[pl.ds(r0, rpt)])
    plsc.subcore_barrier()

    def outer(jb, _):
        c0 = wid * CPW + jb * IB
        pltpu.sync_copy(gidx_hbm.at[pl.ds(c0, IB)], gidx_v)
        pltpu.sync_copy(sidx_hbm.at[pl.ds(c0, IB)], sidx_v)
        gd = [None] * nbuf
        sd = [None] * nbuf
        for k in range(nbuf - 1):
            gd[k] = pltpu.async_copy(vw_hbm.at[gidx_v.at[k]], bufs[k],
                                     gsems[k])
        for q in range(IB):
            x = q % nbuf
            qn = q + nbuf - 1
            if qn < IB:
                y = qn % nbuf
                if sd[y] is not None:
                    sd[y].wait()
                gd[y] = pltpu.async_copy(
                    vw_hbm.at[gidx_v.at[qn]], bufs[y], gsems[y])
            gd[x].wait()
            sd[x] = pltpu.async_copy(
                bufs[x], acc_sh.at[sidx_v.at[q]], ssems[x], add=True)
        for k in range(nbuf):
            if sd[k] is not None:
                sd[k].wait()
        return ()

    lax.fori_loop(0, CPW // IB, outer, ())
    plsc.subcore_barrier()
    pltpu.sync_copy(acc_sh.at[pl.ds(r0, rpt)],
                    out_hbm.at[pl.ds(cid * NPAD + r0, rpt)])


@functools.cache
def _make_scatter(dd, tc_tiling=True, nbuf=2):
    return pl.kernel(
        functools.partial(_scatter_body, nbuf),
        out_type=jax.ShapeDtypeStruct((2 * NPAD, dd), jnp.float32),
        mesh=_get_mesh(),
        compiler_params=pltpu.CompilerParams(use_tc_tiling_on_sc=tc_tiling),
        scratch_types=(
            [pltpu.VMEM((IB, CH), jnp.int32),
             pltpu.VMEM((IB, CH), jnp.int32)]
            + [pltpu.VMEM((CH, dd), jnp.float32)] * nbuf
            + [pltpu.VMEM_SHARED((NPAD, dd), jnp.float32)]
            + [pltpu.SemaphoreType.DMA] * (2 * nbuf)
        ),
    )


BLK = 2048


def _mm_body(n_in, d_out, relu, emit_sum, *refs):
    ins = refs[:n_in]
    w_ref, b_ref = refs[n_in], refs[n_in + 1]
    outs = refs[n_in + 2:]
    x = ins[0][...]
    for r in ins[1:]:
        x = x + r[...]
    if emit_sum:
        outs[2][...] = x
    a = jnp.maximum(x, 0.0) if relu else x
    u = jnp.dot(a, w_ref[...], preferred_element_type=jnp.float32) + b_ref[...]
    h = d_out // 2
    outs[0][...] = u[:, :h]
    outs[1][...] = u[:, h:]


def _combine_matmul(addends, w, b, relu, emit_sum):
    n_in = len(addends)
    d_out = w.shape[1]
    h = d_out // 2
    grid = NPAD // BLK
    in_specs = ([pl.BlockSpec((BLK, D), lambda i: (i, 0)) for _ in addends]
                + [pl.BlockSpec((D, d_out), lambda i: (0, 0)),
                   pl.BlockSpec((1, d_out), lambda i: (0, 0))])
    out_shape = [jax.ShapeDtypeStruct((NPAD, h), jnp.float32),
                 jax.ShapeDtypeStruct((NPAD, h), jnp.float32)]
    out_specs = [pl.BlockSpec((BLK, h), lambda i: (i, 0)),
                 pl.BlockSpec((BLK, h), lambda i: (i, 0))]
    if emit_sum:
        out_shape.append(jax.ShapeDtypeStruct((NPAD, D), jnp.float32))
        out_specs.append(pl.BlockSpec((BLK, D), lambda i: (i, 0)))
    return pl.pallas_call(
        functools.partial(_mm_body, n_in, d_out, relu, emit_sum),
        grid=(grid,),
        in_specs=in_specs,
        out_specs=out_specs,
        out_shape=out_shape,
    )(*addends, w, b.reshape(1, d_out))


def _first_mm_body(img, verts, wci, wcv, bc, wsi, wsv, bs,
                   out1, vw1, sk):
    rv = jnp.maximum(verts[...], 0.0)
    u = (jnp.dot(img[...], wci[...], preferred_element_type=jnp.float32)
         + jnp.dot(rv, wcv[...], preferred_element_type=jnp.float32)
         + bc[...])
    out1[...] = u[:, :D]
    vw1[...] = u[:, D:]
    sk[...] = (jnp.dot(img[...], wsi[...], preferred_element_type=jnp.float32)
               + jnp.dot(verts[...], wsv[...], preferred_element_type=jnp.float32)
               + bs[...])


def _first_mm(img, verts_p, wci, wcv, bc, wsi, wsv, bs):
    grid = NPAD // BLK
    return pl.pallas_call(
        _first_mm_body,
        grid=(grid,),
        in_specs=[
            pl.BlockSpec((BLK, D), lambda i: (i, 0)),
            pl.BlockSpec((BLK, 3), lambda i: (i, 0)),
            pl.BlockSpec((D, 2 * D), lambda i: (0, 0)),
            pl.BlockSpec((3, 2 * D), lambda i: (0, 0)),
            pl.BlockSpec((1, 2 * D), lambda i: (0, 0)),
            pl.BlockSpec((D, D), lambda i: (0, 0)),
            pl.BlockSpec((3, D), lambda i: (0, 0)),
            pl.BlockSpec((1, D), lambda i: (0, 0)),
        ],
        out_specs=[pl.BlockSpec((BLK, D), lambda i: (i, 0))] * 3,
        out_shape=[jax.ShapeDtypeStruct((NPAD, D), jnp.float32)] * 3,
    )(img, verts_p, wci, wcv, bc.reshape(1, 2 * D), wsi, wsv, bs.reshape(1, D))


def _final_body(verts, o16, ma, mb, out):
    t = o16[:, :3] + ma[:, :3] + mb[:, :3]
    out[...] = verts[...] + jnp.tanh(t)


def _final(verts_p, o16, ma, mb):
    grid = NPAD // BLK
    return pl.pallas_call(
        _final_body,
        grid=(grid,),
        in_specs=[
            pl.BlockSpec((BLK, 3), lambda i: (i, 0)),
            pl.BlockSpec((BLK, 16), lambda i: (i, 0)),
            pl.BlockSpec((BLK, 16), lambda i: (i, 0)),
            pl.BlockSpec((BLK, 16), lambda i: (i, 0)),
        ],
        out_specs=pl.BlockSpec((BLK, 3), lambda i: (i, 0)),
        out_shape=jax.ShapeDtypeStruct((NPAD, 3), jnp.float32),
    )(verts_p, o16, ma, mb)


def kernel(feats0, feats1, feats2, feats3, verts, params, edges):
    p = params
    f32 = jnp.float32

    f0 = feats0[0].reshape(256, 3136)
    f1 = feats1[0].reshape(512, 784)
    f2 = jnp.pad(feats2[0].reshape(1024, 196), ((0, 0), (0, 4)))
    f3 = jnp.pad(feats3[0].reshape(2048, 49), ((0, 0), (0, 7)))
    verts_p = jnp.pad(verts, ((0, NPAD - N), (0, 0)))

    s = edges[:, 0].astype(jnp.int32)
    d = edges[:, 1].astype(jnp.int32)
    gidx = jnp.concatenate([d, s])
    sidx = jnp.concatenate([s, d])
    padn = E2P - E2
    gidx = jnp.concatenate(
        [gidx, jnp.arange(padn, dtype=jnp.int32) % N])
    sidx = jnp.concatenate(
        [sidx, N + (jnp.arange(padn, dtype=jnp.int32) % (NPAD - N))])
    gidx = gidx.reshape(NW * CPW, CH)
    sidx = sidx.reshape(NW * CPW, CH)
    zeros128 = jnp.zeros((NPAD, D), f32)

    def wcat(pre):
        return (jnp.concatenate([p[pre + "_w0"], p[pre + "_w1"]], axis=1),
                jnp.concatenate([p[pre + "_b0"], p[pre + "_b1"]]))

    g, idx16, wgt16 = _g_table(f0, f1, f2, f3, p["bottleneck_w"], verts_p)

    img = _get_bilinear()(g, idx16, wgt16, p["bottleneck_b"])

    def scat128(vw):
        n = _make_scatter(D)(vw, gidx, sidx, zeros128)
        return n[:NPAD], n[NPAD:]

    w1c, b1c = wcat("rb0_gc1")
    out1, vw1, sk = _first_mm(img, verts_p, w1c[:D], w1c[D:], b1c,
                              p["rb0_skip_w"][:D], p["rb0_skip_w"][D:],
                              p["rb0_skip_b"])
    na, nb = scat128(vw1)
    w2c, b2c = wcat("rb0_gc2")
    out2, vw2 = _combine_matmul([out1, na, nb], w2c, b2c, True, False)
    na, nb = scat128(vw2)

    w3c, b3c = wcat("rb1_gc1")
    out3, vw3, x1 = _combine_matmul([out2, na, nb, sk], w3c, b3c, True, True)
    na, nb = scat128(vw3)
    w4c, b4c = wcat("rb1_gc2")
    out4, vw4 = _combine_matmul([out3, na, nb], w4c, b4c, True, False)
    na, nb = scat128(vw4)

    w5c, b5c = wcat("rb2_gc1")
    out5, vw5, x2 = _combine_matmul([out4, na, nb, x1], w5c, b5c, True, True)
    na, nb = scat128(vw5)
    w6c, b6c = wcat("rb2_gc2")
    out6, vw6 = _combine_matmul([out5, na, nb], w6c, b6c, True, False)
    na, nb = scat128(vw6)

    wo = jnp.concatenate([
        jnp.pad(p["out_w0"], ((0, 0), (0, 13))),
        jnp.pad(p["out_w1"], ((0, 0), (0, 13)))], axis=1)
    bo = jnp.concatenate([jnp.pad(p["out_b0"], (0, 13)),
                          jnp.pad(p["out_b1"], (0, 13))])
    o16, vwo, x3 = _combine_matmul([out6, na, nb, x2], wo, bo, False, True)
    zeros16 = jnp.zeros((NPAD, 16), jnp.float32)
    m = _make_scatter(16, False, 4)(vwo, gidx, sidx, zeros16)
    new_verts = _final(verts_p, o16, m[:NPAD], m[NPAD:])

    return new_verts[:N], x3[:N]
